# baseline (speedup 1.0000x reference)
import functools

import jax
import jax.numpy as jnp
from jax import lax
from jax.experimental import pallas as pl
from jax.experimental.pallas import tpu as pltpu
from jax.experimental.pallas import tpu_sc as plsc

N = 10000
E = 320000
D = 128
DH = 64
NC = 2
NS = 16
RPT = 624
TAIL = N - NS * RPT

C_B = 80
NCH_B = (E // NS) // C_B
DEPTH = 5
LAG = 2

C_A = 80
NCH_A = (E // NC // NS) // C_A

_mesh = plsc.VectorSubcoreMesh(core_axis_name="c", subcore_axis_name="s")


@functools.partial(
    pl.kernel,
    out_type=jax.ShapeDtypeStruct((NC * N,), jnp.float32),
    mesh=_mesh,
    scratch_types=[
        pltpu.VMEM((NCH_A, C_A), jnp.int32),
        pltpu.VMEM((C_A,), jnp.float32),
        pltpu.VMEM((N,), jnp.float32),
        pltpu.VMEM_SHARED((N,), jnp.float32),
        pltpu.SemaphoreType.DMA,
    ],
)
def _deg_kernel(dst_hbm, zeros_hbm, out_hbm, idx_v, ones_v, deg_v, deg_sh,
                sem):
    c = lax.axis_index("c")
    s = lax.axis_index("s")

    @pl.when(s == 0)
    def _():
        pltpu.sync_copy(zeros_hbm, deg_v)
        pltpu.sync_copy(deg_v, deg_sh)

    pltpu.sync_copy(dst_hbm.at[c, s], idx_v)
    for i in range(C_A // 16):
        ones_v[pl.ds(i * 16, 16)] = jnp.full((16,), 1.0, jnp.float32)
    plsc.subcore_barrier()

    pltpu.async_copy(ones_v, deg_sh.at[idx_v.at[0]], sem, add=True)

    def body(j, carry):
        pltpu.async_copy(ones_v, deg_sh.at[idx_v.at[j]], sem, add=True)
        pltpu.make_async_copy(ones_v, deg_sh.at[idx_v.at[j - 1]], sem).wait()
        return carry

    lax.fori_loop(1, NCH_A, body, 0)
    pltpu.make_async_copy(ones_v, deg_sh.at[idx_v.at[NCH_A - 1]], sem).wait()
    plsc.subcore_barrier()

    @pl.when(s == 0)
    def _():
        pltpu.sync_copy(deg_sh, deg_v)
        pltpu.sync_copy(deg_v, out_hbm.at[pl.ds(c * N, N)])


@functools.partial(
    pl.kernel,
    out_type=jax.ShapeDtypeStruct((NC, N, DH), jnp.float32),
    mesh=_mesh,
    scratch_types=[
        pltpu.VMEM((NCH_B, C_B), jnp.int32),
        pltpu.VMEM((NCH_B, C_B), jnp.int32),
        [pltpu.VMEM((C_B, DH), jnp.float32) for _ in range(DEPTH)],
        pltpu.VMEM_SHARED((N, DH), jnp.float32),
        pltpu.SemaphoreType.DMA((DEPTH,)),
        pltpu.SemaphoreType.DMA((DEPTH,)),
    ],
    compiler_params=pltpu.CompilerParams(use_tc_tiling_on_sc=False),
)
def _scatter_kernel(h2_hbm, src_hbm, dst_hbm, zeros_hbm, out_hbm,
                    src_v, dst_v, bufs, agg_sh, gsems, ssems):
    c = lax.axis_index("c")
    s = lax.axis_index("s")

    pltpu.sync_copy(zeros_hbm.at[pl.ds(s * RPT, RPT)],
                    agg_sh.at[pl.ds(s * RPT, RPT)])

    @pl.when(s == 0)
    def _():
        pltpu.sync_copy(zeros_hbm.at[pl.ds(NS * RPT, TAIL)],
                        agg_sh.at[pl.ds(NS * RPT, TAIL)])

    pltpu.sync_copy(src_hbm.at[c, s], src_v)
    pltpu.sync_copy(dst_hbm.at[s], dst_v)
    plsc.subcore_barrier()

    def wait_g(g, b):
        pltpu.make_async_copy(h2_hbm.at[src_v.at[g]], bufs[b],
                              gsems.at[b]).wait()

    def issue_s(g, b):
        pltpu.async_copy(bufs[b], agg_sh.at[dst_v.at[g]],
                         ssems.at[b], add=True)

    def wait_s(g, b):
        pltpu.make_async_copy(bufs[b], agg_sh.at[dst_v.at[g]],
                              ssems.at[b]).wait()

    def issue_g(g, b):
        pltpu.async_copy(h2_hbm.at[src_v.at[g]], bufs[b], gsems.at[b])

    def step(g, b, do_wait_s, do_issue_g):
        wait_g(g, b)
        issue_s(g, b)
        bl = (b - LAG) % DEPTH
        if do_wait_s:
            wait_s(g - LAG, bl)
        if do_issue_g:
            issue_g(g + DEPTH - LAG, bl)

    for b in range(DEPTH - LAG):
        issue_g(b, b)
    for b in range(DEPTH):
        step(b, b, b >= LAG, b + DEPTH - LAG < NCH_B)

    def group(q, carry):
        g0 = q * DEPTH
        for b in range(DEPTH):
            step(g0 + b, b, True, True)
        return carry

    lax.fori_loop(1, NCH_B // DEPTH - 1, group, 0)
    for b in range(DEPTH):
        g = NCH_B - DEPTH + b
        step(g, b, True, g + DEPTH - LAG < NCH_B)
    for i in range(LAG):
        g = NCH_B - LAG + i
        wait_s(g, g % DEPTH)
    plsc.subcore_barrier()

    pltpu.sync_copy(agg_sh.at[pl.ds(s * RPT, RPT)],
                    out_hbm.at[c, pl.ds(s * RPT, RPT)])

    @pl.when(s == 0)
    def _():
        pltpu.sync_copy(agg_sh.at[pl.ds(NS * RPT, TAIL)],
                        out_hbm.at[c, pl.ds(NS * RPT, TAIL)])


_ROWS_BLK = 2000
_GRID = N // _ROWS_BLK


def _h2_body(x_ref, w_ref, degp_ref, h2_ref):
    deg = degp_ref[0] + degp_ref[1] + 1.0
    dis = lax.rsqrt(deg)
    h = jnp.dot(x_ref[...], w_ref[...], preferred_element_type=jnp.float32)
    h2 = h * dis
    h2_ref[0] = h2[:, :DH]
    h2_ref[1] = h2[:, DH:]


def _heads_body(aggp_ref, h2p_ref, degp_ref, b_ref, wmu_ref, bmu_ref,
                wlv_ref, blv_ref, mu_ref, lv_ref):
    deg = degp_ref[0] + degp_ref[1] + 1.0
    dis = lax.rsqrt(deg)
    pre = jnp.concatenate(
        [aggp_ref[0] + h2p_ref[0], aggp_ref[1] + h2p_ref[1]], axis=1)
    z = jnp.maximum(pre * dis + b_ref[...], 0.0)
    mu_ref[...] = jnp.dot(z, wmu_ref[...],
                          preferred_element_type=jnp.float32) + bmu_ref[...]
    lv_ref[...] = jnp.dot(z, wlv_ref[...],
                          preferred_element_type=jnp.float32) + blv_ref[...]


def _tc_h2(x, W, deg_parts):
    return pl.pallas_call(
        _h2_body,
        grid=(_GRID,),
        in_specs=[
            pl.BlockSpec((_ROWS_BLK, D), lambda i: (i, 0)),
            pl.BlockSpec((D, D), lambda i: (0, 0)),
            pl.BlockSpec((NC, _ROWS_BLK, 1), lambda i: (0, i, 0)),
        ],
        out_specs=pl.BlockSpec((NC, _ROWS_BLK, DH), lambda i: (0, i, 0)),
        out_shape=jax.ShapeDtypeStruct((NC, N, DH), jnp.float32),
    )(x, W, deg_parts)


def _tc_heads(agg, h2, deg_parts, b, Wmu, bmu, Wlv, blv):
    return pl.pallas_call(
        _heads_body,
        grid=(_GRID,),
        in_specs=[
            pl.BlockSpec((NC, _ROWS_BLK, DH), lambda i: (0, i, 0)),
            pl.BlockSpec((NC, _ROWS_BLK, DH), lambda i: (0, i, 0)),
            pl.BlockSpec((NC, _ROWS_BLK, 1), lambda i: (0, i, 0)),
            pl.BlockSpec((1, D), lambda i: (0, 0)),
            pl.BlockSpec((D, D), lambda i: (0, 0)),
            pl.BlockSpec((1, D), lambda i: (0, 0)),
            pl.BlockSpec((D, D), lambda i: (0, 0)),
            pl.BlockSpec((1, D), lambda i: (0, 0)),
        ],
        out_specs=[
            pl.BlockSpec((_ROWS_BLK, D), lambda i: (i, 0)),
            pl.BlockSpec((_ROWS_BLK, D), lambda i: (i, 0)),
        ],
        out_shape=[
            jax.ShapeDtypeStruct((N, D), jnp.float32),
            jax.ShapeDtypeStruct((N, D), jnp.float32),
        ],
    )(agg, h2, deg_parts, b, Wmu, bmu, Wlv, blv)


def kernel(x, edge_index, W, b, Wmu, bmu, Wlv, blv):
    src = edge_index[0].astype(jnp.int32)
    dst = edge_index[1].astype(jnp.int32)

    zeros_n = jnp.zeros((N,), jnp.float32)
    deg_flat = _deg_kernel(dst.reshape(NC, NS, NCH_A, C_A), zeros_n)

    degp3 = deg_flat.reshape(NC, N, 1)
    h2 = _tc_h2(x, W, degp3)

    zeros_nd = jnp.zeros((N, DH), jnp.float32)
    src_r = src.reshape(NS, NCH_B, C_B)
    src2 = jnp.stack([src_r, src_r + N])
    agg = _scatter_kernel(h2.reshape(NC * N, DH), src2,
                          dst.reshape(NS, NCH_B, C_B), zeros_nd)

    mu, lv = _tc_heads(agg, h2, degp3, b.reshape(1, D), Wmu,
                       bmu.reshape(1, D), Wlv, blv.reshape(1, D))
    return (mu, lv)


# speedup vs baseline: 1.0640x; 1.0640x over previous
import functools

import jax
import jax.numpy as jnp
from jax import lax
from jax.experimental import pallas as pl
from jax.experimental.pallas import tpu as pltpu
from jax.experimental.pallas import tpu_sc as plsc

N = 10000
E = 320000
D = 128
DH = 64
NC = 2
NS = 16
RPT = 624
TAIL = N - NS * RPT

C_B = 80
NCH_B = (E // NS) // C_B
DEPTH = 5
LAG = 2

C_A = 80
NCH_A = (E // NC // NS) // C_A

_mesh = plsc.VectorSubcoreMesh(core_axis_name="c", subcore_axis_name="s")


@functools.partial(
    pl.kernel,
    out_type=jax.ShapeDtypeStruct((NC * N,), jnp.float32),
    mesh=_mesh,
    scratch_types=[
        pltpu.VMEM((NCH_A, C_A), jnp.int32),
        pltpu.VMEM((C_A,), jnp.float32),
        pltpu.VMEM((N,), jnp.float32),
        pltpu.VMEM_SHARED((N,), jnp.float32),
        pltpu.SemaphoreType.DMA,
    ],
)
def _deg_kernel(dst_hbm, zeros_hbm, out_hbm, idx_v, ones_v, deg_v, deg_sh,
                sem):
    c = lax.axis_index("c")
    s = lax.axis_index("s")

    @pl.when(s == 0)
    def _():
        pltpu.sync_copy(zeros_hbm, deg_v)
        pltpu.sync_copy(deg_v, deg_sh)

    pltpu.sync_copy(dst_hbm.at[c, s], idx_v)
    for i in range(C_A // 16):
        ones_v[pl.ds(i * 16, 16)] = jnp.full((16,), 1.0, jnp.float32)
    plsc.subcore_barrier()

    pltpu.async_copy(ones_v, deg_sh.at[idx_v.at[0]], sem, add=True)

    def body(j, carry):
        pltpu.async_copy(ones_v, deg_sh.at[idx_v.at[j]], sem, add=True)
        pltpu.make_async_copy(ones_v, deg_sh.at[idx_v.at[j - 1]], sem).wait()
        return carry

    lax.fori_loop(1, NCH_A, body, 0)
    pltpu.make_async_copy(ones_v, deg_sh.at[idx_v.at[NCH_A - 1]], sem).wait()
    plsc.subcore_barrier()

    @pl.when(s == 0)
    def _():
        pltpu.sync_copy(deg_sh, deg_v)
        pltpu.sync_copy(deg_v, out_hbm.at[pl.ds(c * N, N)])


@functools.partial(
    pl.kernel,
    out_type=jax.ShapeDtypeStruct((NC, N, DH), jnp.float32),
    mesh=_mesh,
    scratch_types=[
        pltpu.VMEM((NCH_B, C_B), jnp.int32),
        pltpu.VMEM((NCH_B, C_B), jnp.int32),
        [pltpu.VMEM((C_B, DH), jnp.float32) for _ in range(DEPTH)],
        pltpu.VMEM_SHARED((N, DH), jnp.float32),
        pltpu.SemaphoreType.DMA((DEPTH,)),
        pltpu.SemaphoreType.DMA((DEPTH,)),
    ],
    compiler_params=pltpu.CompilerParams(use_tc_tiling_on_sc=False),
)
def _scatter_kernel(h2_hbm, src_hbm, dst_hbm, zeros_hbm, out_hbm,
                    src_v, dst_v, bufs, agg_sh, gsems, ssems):
    c = lax.axis_index("c")
    s = lax.axis_index("s")

    pltpu.sync_copy(zeros_hbm.at[pl.ds(s * RPT, RPT)],
                    agg_sh.at[pl.ds(s * RPT, RPT)])

    @pl.when(s == 0)
    def _():
        pltpu.sync_copy(zeros_hbm.at[pl.ds(NS * RPT, TAIL)],
                        agg_sh.at[pl.ds(NS * RPT, TAIL)])

    pltpu.sync_copy(src_hbm.at[c, s], src_v)
    pltpu.sync_copy(dst_hbm.at[s], dst_v)
    plsc.subcore_barrier()

    def wait_g(g, b):
        pltpu.make_async_copy(h2_hbm.at[src_v.at[g]], bufs[b],
                              gsems.at[b]).wait()

    def issue_s(g, b):
        pltpu.async_copy(bufs[b], agg_sh.at[dst_v.at[g]],
                         ssems.at[b], add=True)

    def wait_s(g, b):
        pltpu.make_async_copy(bufs[b], agg_sh.at[dst_v.at[g]],
                              ssems.at[b]).wait()

    def issue_g(g, b):
        pltpu.async_copy(h2_hbm.at[src_v.at[g]], bufs[b], gsems.at[b])

    def step(g, b, do_issue_g):
        wait_g(g, b)
        issue_s(g, b)
        wait_s(g, b)
        if do_issue_g:
            issue_g(g + DEPTH, b)

    for b in range(DEPTH):
        issue_g(b, b)
    for b in range(DEPTH):
        step(b, b, True)

    def group(q, carry):
        g0 = q * DEPTH
        for b in range(DEPTH):
            step(g0 + b, b, True)
        return carry

    lax.fori_loop(1, NCH_B // DEPTH - 1, group, 0)
    for b in range(DEPTH):
        step(NCH_B - DEPTH + b, b, False)
    plsc.subcore_barrier()

    pltpu.sync_copy(agg_sh.at[pl.ds(s * RPT, RPT)],
                    out_hbm.at[c, pl.ds(s * RPT, RPT)])

    @pl.when(s == 0)
    def _():
        pltpu.sync_copy(agg_sh.at[pl.ds(NS * RPT, TAIL)],
                        out_hbm.at[c, pl.ds(NS * RPT, TAIL)])


_ROWS_BLK = 2000
_GRID = N // _ROWS_BLK


def _h2_body(x_ref, w_ref, degp_ref, h2_ref):
    deg = degp_ref[0] + degp_ref[1] + 1.0
    dis = lax.rsqrt(deg)
    h = jnp.dot(x_ref[...], w_ref[...], preferred_element_type=jnp.float32)
    h2 = h * dis
    h2_ref[0] = h2[:, :DH]
    h2_ref[1] = h2[:, DH:]


def _heads_body(aggp_ref, h2p_ref, degp_ref, b_ref, wmu_ref, bmu_ref,
                wlv_ref, blv_ref, mu_ref, lv_ref):
    deg = degp_ref[0] + degp_ref[1] + 1.0
    dis = lax.rsqrt(deg)
    pre = jnp.concatenate(
        [aggp_ref[0] + h2p_ref[0], aggp_ref[1] + h2p_ref[1]], axis=1)
    z = jnp.maximum(pre * dis + b_ref[...], 0.0)
    mu_ref[...] = jnp.dot(z, wmu_ref[...],
                          preferred_element_type=jnp.float32) + bmu_ref[...]
    lv_ref[...] = jnp.dot(z, wlv_ref[...],
                          preferred_element_type=jnp.float32) + blv_ref[...]


def _tc_h2(x, W, deg_parts):
    return pl.pallas_call(
        _h2_body,
        grid=(_GRID,),
        in_specs=[
            pl.BlockSpec((_ROWS_BLK, D), lambda i: (i, 0)),
            pl.BlockSpec((D, D), lambda i: (0, 0)),
            pl.BlockSpec((NC, _ROWS_BLK, 1), lambda i: (0, i, 0)),
        ],
        out_specs=pl.BlockSpec((NC, _ROWS_BLK, DH), lambda i: (0, i, 0)),
        out_shape=jax.ShapeDtypeStruct((NC, N, DH), jnp.float32),
    )(x, W, deg_parts)


def _tc_heads(agg, h2, deg_parts, b, Wmu, bmu, Wlv, blv):
    return pl.pallas_call(
        _heads_body,
        grid=(_GRID,),
        in_specs=[
            pl.BlockSpec((NC, _ROWS_BLK, DH), lambda i: (0, i, 0)),
            pl.BlockSpec((NC, _ROWS_BLK, DH), lambda i: (0, i, 0)),
            pl.BlockSpec((NC, _ROWS_BLK, 1), lambda i: (0, i, 0)),
            pl.BlockSpec((1, D), lambda i: (0, 0)),
            pl.BlockSpec((D, D), lambda i: (0, 0)),
            pl.BlockSpec((1, D), lambda i: (0, 0)),
            pl.BlockSpec((D, D), lambda i: (0, 0)),
            pl.BlockSpec((1, D), lambda i: (0, 0)),
        ],
        out_specs=[
            pl.BlockSpec((_ROWS_BLK, D), lambda i: (i, 0)),
            pl.BlockSpec((_ROWS_BLK, D), lambda i: (i, 0)),
        ],
        out_shape=[
            jax.ShapeDtypeStruct((N, D), jnp.float32),
            jax.ShapeDtypeStruct((N, D), jnp.float32),
        ],
    )(agg, h2, deg_parts, b, Wmu, bmu, Wlv, blv)


def kernel(x, edge_index, W, b, Wmu, bmu, Wlv, blv):
    src = edge_index[0].astype(jnp.int32)
    dst = edge_index[1].astype(jnp.int32)

    zeros_n = jnp.zeros((N,), jnp.float32)
    deg_flat = _deg_kernel(dst.reshape(NC, NS, NCH_A, C_A), zeros_n)

    degp3 = deg_flat.reshape(NC, N, 1)
    h2 = _tc_h2(x, W, degp3)

    zeros_nd = jnp.zeros((N, DH), jnp.float32)
    src_r = src.reshape(NS, NCH_B, C_B)
    src2 = jnp.stack([src_r, src_r + N])
    agg = _scatter_kernel(h2.reshape(NC * N, DH), src2,
                          dst.reshape(NS, NCH_B, C_B), zeros_nd)

    mu, lv = _tc_heads(agg, h2, degp3, b.reshape(1, D), Wmu,
                       bmu.reshape(1, D), Wlv, blv.reshape(1, D))
    return (mu, lv)
